# baseline (speedup 1.0000x reference)
import functools

import jax
import jax.numpy as jnp
from jax import lax
from jax.experimental import pallas as pl
from jax.experimental.pallas import tpu as pltpu
from jax.experimental.pallas import tpu_sc as plsc

_N = 10000
_E = 320000
_D = 128
_DO = 16
_NW = 32
_EPW = _E // _NW
_L = 16
_T = 2048
_NPAD = 10240


_EC = 9984
_ER = _E - _NW * _EC


def _hist_body(edge_hbm, out_hbm, idx_v, rem_v, cnt_v, cnt2_v, sem):
    wid = lax.axis_index("s") * 2 + lax.axis_index("c")
    cp = pltpu.make_async_copy(
        edge_hbm.at[:, pl.ds(wid * _EC, _EC)], idx_v, sem)
    cp.start()
    rcp = pltpu.make_async_copy(
        edge_hbm.at[:, pl.ds(_NW * _EC, _ER)], rem_v, sem)

    @pl.when(wid == _NW - 1)
    def _():
        rcp.start()

    zeros = jnp.zeros((_L,), jnp.float32)

    def zero_body(i, carry):
        cnt_v[pl.ds(i * _L, _L)] = zeros
        cnt2_v[pl.ds(i * _L, _L)] = zeros
        return carry

    lax.fori_loop(0, _NPAD // _L, zero_body, 0, unroll=8)
    cp.wait()

    ones = jnp.ones((_L,), jnp.float32)

    def scat_body(i, carry):
        idx = idx_v[0, pl.ds(2 * i * _L, _L)]
        plsc.addupdate_scatter(cnt_v, [idx], ones)
        idx2 = idx_v[0, pl.ds((2 * i + 1) * _L, _L)]
        plsc.addupdate_scatter(cnt2_v, [idx2], ones)
        return carry

    lax.fori_loop(0, _EC // (2 * _L), scat_body, 0, unroll=4)

    @pl.when(wid == _NW - 1)
    def _():
        rcp.wait()

        def rem_body(i, carry):
            idx = rem_v[0, pl.ds(2 * i * _L, _L)]
            plsc.addupdate_scatter(cnt_v, [idx], ones)
            idx2 = rem_v[0, pl.ds((2 * i + 1) * _L, _L)]
            plsc.addupdate_scatter(cnt2_v, [idx2], ones)
            return carry

        lax.fori_loop(0, _ER // (2 * _L), rem_body, 0, unroll=4)

    def merge_body(i, carry):
        sl = pl.ds(i * _L, _L)
        cnt_v[sl] = cnt_v[sl] + cnt2_v[sl]
        return carry

    lax.fori_loop(0, _NPAD // _L, merge_body, 0, unroll=8)
    pltpu.sync_copy(cnt_v, out_hbm.at[wid])


_hist = pl.kernel(
    _hist_body,
    out_type=jax.ShapeDtypeStruct((_NW, _NPAD), jnp.float32),
    mesh=plsc.VectorSubcoreMesh(core_axis_name="c", subcore_axis_name="s"),
    scratch_types=[
        pltpu.VMEM((2, _EC), jnp.int32),
        pltpu.VMEM((2, _ER), jnp.int32),
        pltpu.VMEM((_NPAD,), jnp.float32),
        pltpu.VMEM((_NPAD,), jnp.float32),
        pltpu.SemaphoreType.DMA,
    ],
    compiler_params=pltpu.CompilerParams(needs_layout_passes=False),
)


def _enc_body(x_ref, cnt_ref, w1_ref, b1_ref, w2_ref, b2_ref, wp_ref,
              bp_ref, out_ref, acc_ref):
    i = pl.program_id(0)

    @pl.when(i == 0)
    def _():
        acc_ref[...] = jnp.zeros_like(acc_ref)

    a = jnp.dot(x_ref[...], w1_ref[...], preferred_element_type=jnp.float32)
    a = jnp.maximum(a + b1_ref[...], 0.0)
    u = jnp.sum(cnt_ref[...], axis=0, keepdims=True)
    col = lax.broadcasted_iota(jnp.int32, (1, _T), 1) + i * _T
    u = u + jnp.where(col < _N, 1.0, 0.0)
    acc_ref[...] += jnp.dot(u, a, preferred_element_type=jnp.float32)

    @pl.when(i == pl.num_programs(0) - 1)
    def _():
        pooled = jnp.dot(acc_ref[...], w2_ref[...],
                         preferred_element_type=jnp.float32)
        pooled = (pooled + float(_N + _E) * b2_ref[...]) * (1.0 / _N)
        out_ref[...] = jnp.dot(pooled, wp_ref[...],
                               preferred_element_type=jnp.float32) + bp_ref[...]


def _encode(x_pad, counts, W1, b1, W2, b2, Wp, bp):
    return pl.pallas_call(
        _enc_body,
        grid=(_NPAD // _T,),
        in_specs=[
            pl.BlockSpec((_T, _D), lambda i: (i, 0)),
            pl.BlockSpec((_NW, _T), lambda i: (0, i)),
            pl.BlockSpec((_D, _D), lambda i: (0, 0)),
            pl.BlockSpec((1, _D), lambda i: (0, 0)),
            pl.BlockSpec((_D, _D), lambda i: (0, 0)),
            pl.BlockSpec((1, _D), lambda i: (0, 0)),
            pl.BlockSpec((_D, _DO), lambda i: (0, 0)),
            pl.BlockSpec((1, _DO), lambda i: (0, 0)),
        ],
        out_specs=pl.BlockSpec((1, _DO), lambda i: (0, 0)),
        out_shape=jax.ShapeDtypeStruct((1, _DO), jnp.float32),
        scratch_shapes=[pltpu.VMEM((1, _D), jnp.float32)],
    )(x_pad, counts, W1, b1, W2, b2, Wp, bp)


def kernel(node_features, edge_index, W1, b1, W2, b2, Wp, bp):
    counts = _hist(edge_index.astype(jnp.int32))
    x_pad = jnp.zeros((_NPAD, _D), jnp.float32).at[:_N].set(node_features)
    out = _encode(x_pad, counts, W1, b1.reshape(1, _D), W2,
                  b2.reshape(1, _D), Wp, bp.reshape(1, _DO))
    return out.reshape(_DO)


# speedup vs baseline: 1.1971x; 1.1971x over previous
import functools

import jax
import jax.numpy as jnp
from jax import lax
from jax.experimental import pallas as pl
from jax.experimental.pallas import tpu as pltpu
from jax.experimental.pallas import tpu_sc as plsc

_N = 10000
_E = 320000
_D = 128
_DO = 16
_NW = 32
_EPW = _E // _NW
_L = 16
_T = 2048
_NPAD = 10240


_EC = 9984
_ER = _E - _NW * _EC


def _hist_body(edge_hbm, out_hbm, idx_v, rem_v, cnt_v, sem):
    wid = lax.axis_index("s") * 2 + lax.axis_index("c")
    cp = pltpu.make_async_copy(
        edge_hbm.at[:, pl.ds(wid * _EC, _EC)], idx_v, sem)
    cp.start()
    rcp = pltpu.make_async_copy(
        edge_hbm.at[:, pl.ds(_NW * _EC, _ER)], rem_v, sem)

    @pl.when(wid == _NW - 1)
    def _():
        rcp.start()

    zeros = jnp.zeros((_L,), jnp.float32)

    def zero_body(i, carry):
        cnt_v[pl.ds(i * _L, _L)] = zeros
        return carry

    lax.fori_loop(0, _NPAD // _L, zero_body, 0, unroll=16)
    cp.wait()

    ones = jnp.ones((_L,), jnp.float32)

    def scat_body(i, carry):
        idx = idx_v[0, pl.ds(i * _L, _L)]
        plsc.addupdate_scatter(cnt_v, [idx], ones)
        return carry

    lax.fori_loop(0, _EC // _L, scat_body, 0, unroll=8)

    @pl.when(wid == _NW - 1)
    def _():
        rcp.wait()

        def rem_body(i, carry):
            idx = rem_v[0, pl.ds(i * _L, _L)]
            plsc.addupdate_scatter(cnt_v, [idx], ones)
            return carry

        lax.fori_loop(0, _ER // _L, rem_body, 0, unroll=8)

    pltpu.sync_copy(cnt_v, out_hbm.at[wid])


_hist = pl.kernel(
    _hist_body,
    out_type=jax.ShapeDtypeStruct((_NW, _NPAD), jnp.float32),
    mesh=plsc.VectorSubcoreMesh(core_axis_name="c", subcore_axis_name="s"),
    scratch_types=[
        pltpu.VMEM((2, _EC), jnp.int32),
        pltpu.VMEM((2, _ER), jnp.int32),
        pltpu.VMEM((_NPAD,), jnp.float32),
        pltpu.SemaphoreType.DMA,
    ],
    compiler_params=pltpu.CompilerParams(needs_layout_passes=False),
)


_TA = 2000


def _enc_a_body(x_ref, w1_ref, b1_ref, a_ref, s1_ref, acc_ref):
    i = pl.program_id(0)

    @pl.when(i == 0)
    def _():
        acc_ref[...] = jnp.zeros_like(acc_ref)

    a = jnp.dot(x_ref[...], w1_ref[...], preferred_element_type=jnp.float32)
    a = jnp.maximum(a + b1_ref[...], 0.0)
    a_ref[...] = a
    acc_ref[...] += jnp.sum(a, axis=0, keepdims=True)

    @pl.when(i == pl.num_programs(0) - 1)
    def _():
        s1_ref[...] = acc_ref[...]


def _enc_a(x, W1, b1):
    return pl.pallas_call(
        _enc_a_body,
        grid=(_N // _TA,),
        in_specs=[
            pl.BlockSpec((_TA, _D), lambda i: (i, 0)),
            pl.BlockSpec((_D, _D), lambda i: (0, 0)),
            pl.BlockSpec((1, _D), lambda i: (0, 0)),
        ],
        out_specs=[
            pl.BlockSpec((_TA, _D), lambda i: (i, 0)),
            pl.BlockSpec((1, _D), lambda i: (0, 0)),
        ],
        out_shape=[
            jax.ShapeDtypeStruct((_N, _D), jnp.float32),
            jax.ShapeDtypeStruct((1, _D), jnp.float32),
        ],
        scratch_shapes=[pltpu.VMEM((1, _D), jnp.float32)],
    )(x, W1, b1)


def _enc_b_body(cnt_ref, a_ref, s1_ref, w2_ref, b2_ref, wp_ref, bp_ref,
                out_ref):
    c = jnp.sum(cnt_ref[...], axis=0, keepdims=True)
    v = jnp.dot(c[:, :_N], a_ref[...],
                preferred_element_type=jnp.float32) + s1_ref[...]
    pooled = jnp.dot(v, w2_ref[...], preferred_element_type=jnp.float32)
    pooled = (pooled + float(_N + _E) * b2_ref[...]) * (1.0 / _N)
    out_ref[...] = jnp.dot(pooled, wp_ref[...],
                           preferred_element_type=jnp.float32) + bp_ref[...]


def _enc_b(counts, a, s1, W2, b2, Wp, bp):
    return pl.pallas_call(
        _enc_b_body,
        out_shape=jax.ShapeDtypeStruct((1, _DO), jnp.float32),
    )(counts, a, s1, W2, b2, Wp, bp)


def kernel(node_features, edge_index, W1, b1, W2, b2, Wp, bp):
    counts = _hist(edge_index.astype(jnp.int32))
    a, s1 = _enc_a(node_features, W1, b1.reshape(1, _D))
    out = _enc_b(counts, a, s1, W2, b2.reshape(1, _D), Wp,
                 bp.reshape(1, _DO))
    return out.reshape(_DO)


# speedup vs baseline: 1.2313x; 1.0286x over previous
import functools

import jax
import jax.numpy as jnp
from jax import lax
from jax.experimental import pallas as pl
from jax.experimental.pallas import tpu as pltpu
from jax.experimental.pallas import tpu_sc as plsc

_N = 10000
_E = 320000
_D = 128
_DO = 16
_NW = 32
_EPW = _E // _NW
_L = 16
_T = 2048
_NPAD = 10240


_EC = 9984
_ER = _E - _NW * _EC


def _hist_body(edge_hbm, out_hbm, idx_v, rem_v, cnt_v, sem):
    wid = lax.axis_index("s") * 2 + lax.axis_index("c")
    cp = pltpu.make_async_copy(
        edge_hbm.at[:, pl.ds(wid * _EC, _EC)], idx_v, sem)
    cp.start()
    rcp = pltpu.make_async_copy(
        edge_hbm.at[:, pl.ds(_NW * _EC, _ER)], rem_v, sem)

    @pl.when(wid == _NW - 1)
    def _():
        rcp.start()

    zeros = jnp.zeros((_L,), jnp.float32)

    def zero_body(i, carry):
        cnt_v[pl.ds(i * _L, _L)] = zeros
        return carry

    lax.fori_loop(0, _NPAD // _L, zero_body, 0, unroll=16)
    cp.wait()

    ones = jnp.ones((_L,), jnp.float32)

    def scat_body(i, carry):
        idx = idx_v[0, pl.ds(i * _L, _L)]
        plsc.addupdate_scatter(cnt_v, [idx], ones)
        return carry

    lax.fori_loop(0, _EC // _L, scat_body, 0, unroll=8)

    @pl.when(wid == _NW - 1)
    def _():
        rcp.wait()

        def rem_body(i, carry):
            idx = rem_v[0, pl.ds(i * _L, _L)]
            plsc.addupdate_scatter(cnt_v, [idx], ones)
            return carry

        lax.fori_loop(0, _ER // _L, rem_body, 0, unroll=8)

    pltpu.sync_copy(cnt_v, out_hbm.at[wid])


_hist = pl.kernel(
    _hist_body,
    out_type=jax.ShapeDtypeStruct((_NW, _NPAD), jnp.float32),
    mesh=plsc.VectorSubcoreMesh(core_axis_name="c", subcore_axis_name="s"),
    scratch_types=[
        pltpu.VMEM((2, _EC), jnp.int32),
        pltpu.VMEM((2, _ER), jnp.int32),
        pltpu.VMEM((_NPAD,), jnp.float32),
        pltpu.SemaphoreType.DMA,
    ],
    compiler_params=pltpu.CompilerParams(needs_layout_passes=False),
)


_TA = 2000


def _enc_a_body(x_ref, w1_ref, b1_ref, a_ref, s1_ref, acc_ref):
    i = pl.program_id(0)

    @pl.when(i == 0)
    def _():
        acc_ref[...] = jnp.zeros_like(acc_ref)

    a = jnp.dot(x_ref[...], w1_ref[...], preferred_element_type=jnp.float32)
    a = jnp.maximum(a + b1_ref[...], 0.0)
    a_ref[...] = a.astype(jnp.bfloat16)
    acc_ref[...] += jnp.sum(a, axis=0, keepdims=True)

    @pl.when(i == pl.num_programs(0) - 1)
    def _():
        s1_ref[...] = acc_ref[...]


def _enc_a(x, W1, b1):
    return pl.pallas_call(
        _enc_a_body,
        grid=(_N // _TA,),
        in_specs=[
            pl.BlockSpec((_TA, _D), lambda i: (i, 0)),
            pl.BlockSpec((_D, _D), lambda i: (0, 0)),
            pl.BlockSpec((1, _D), lambda i: (0, 0)),
        ],
        out_specs=[
            pl.BlockSpec((_TA, _D), lambda i: (i, 0)),
            pl.BlockSpec((1, _D), lambda i: (0, 0)),
        ],
        out_shape=[
            jax.ShapeDtypeStruct((_N, _D), jnp.bfloat16),
            jax.ShapeDtypeStruct((1, _D), jnp.float32),
        ],
        scratch_shapes=[pltpu.VMEM((1, _D), jnp.float32)],
    )(x, W1, b1)


def _enc_b_body(cnt_ref, a_ref, s1_ref, w2_ref, b2_ref, wp_ref, bp_ref,
                out_ref):
    c = jnp.sum(cnt_ref[...], axis=0, keepdims=True)
    v = jnp.dot(c[:, :_N].astype(jnp.bfloat16), a_ref[...],
                preferred_element_type=jnp.float32) + s1_ref[...]
    pooled = jnp.dot(v, w2_ref[...], preferred_element_type=jnp.float32)
    pooled = (pooled + float(_N + _E) * b2_ref[...]) * (1.0 / _N)
    out_ref[...] = jnp.dot(pooled, wp_ref[...],
                           preferred_element_type=jnp.float32) + bp_ref[...]


def _enc_b(counts, a, s1, W2, b2, Wp, bp):
    return pl.pallas_call(
        _enc_b_body,
        out_shape=jax.ShapeDtypeStruct((1, _DO), jnp.float32),
    )(counts, a, s1, W2, b2, Wp, bp)


def kernel(node_features, edge_index, W1, b1, W2, b2, Wp, bp):
    counts = _hist(edge_index.astype(jnp.int32))
    a, s1 = _enc_a(node_features, W1, b1.reshape(1, _D))
    out = _enc_b(counts, a, s1, W2, b2.reshape(1, _D), Wp,
                 bp.reshape(1, _DO))
    return out.reshape(_DO)
